# baseline (speedup 1.0000x reference)
import functools

import jax
import jax.numpy as jnp
from jax import lax
from jax.experimental import pallas as pl
from jax.experimental.pallas import tpu as pltpu
from jax.experimental.pallas import tpu_sc as plsc

N = 10000
E = 320000
D = 128
ED = 16
L = 4
C = 20

NC = 2
NS = 16
NW = NC * NS
EPW = E // NW
K = 80
STEPS = EPW // K
NB = 5
SUPK = NB * K
NSUP = STEPS // NB
SK = 80
SSTEPS = EPW // SK
SG = 1
SROWS = SG * SK
SGROUPS = SSTEPS // SG
ZR = 624
ZR_LAST = N - (NS - 1) * ZR

BN = 2000
BE = 16000


def _warm_body(in_hbm, out_hbm, buf_v):
    @pl.when((lax.axis_index("s") == 0) & (lax.axis_index("c") == 0))
    def _():
        pltpu.sync_copy(in_hbm, buf_v)
        pltpu.sync_copy(buf_v, out_hbm)


def _sc_warm(x8):
    mesh = plsc.VectorSubcoreMesh(core_axis_name="c", subcore_axis_name="s", num_cores=NC, num_subcores=NS)
    f = pl.kernel(
        _warm_body,
        out_type=jax.ShapeDtypeStruct((8,), jnp.int32),
        mesh=mesh,
        scratch_types=[pltpu.VMEM((8,), jnp.int32)],
    )
    return f(x8)


def _gather_body(g_hbm, src_hbm, warm_hbm, out_hbm, idx_v, buf0, buf1,
                 sem_g0, sem_g1, sem_w):
    del warm_hbm
    wid = lax.axis_index("s") * NC + lax.axis_index("c")
    base = wid * EPW
    pltpu.sync_copy(src_hbm.at[wid], idx_v)

    def issue_gathers(s, buf, sem):
        for b in range(NB):
            j = s * NB + b
            pltpu.async_copy(g_hbm.at[idx_v.at[j]],
                             buf.at[pl.ds(b * K, K)], sem)

    def drain_gathers(buf, sem):
        for b in range(NB):
            pltpu.make_async_copy(g_hbm.at[idx_v.at[0]],
                                  buf.at[pl.ds(b * K, K)], sem).wait()

    def drain_wb(buf):
        pltpu.make_async_copy(buf, out_hbm.at[pl.ds(0, SUPK)], sem_w).wait()

    issue_gathers(0, buf0, sem_g0)

    def do_super(s, cur, nxt, sem_cur, sem_nxt):
        @pl.when(s >= 1)
        def _():
            drain_wb(nxt)

        @pl.when(s + 1 < NSUP)
        def _():
            issue_gathers(s + 1, nxt, sem_nxt)

        drain_gathers(cur, sem_cur)
        off = pl.multiple_of(base + s * SUPK, 8)
        pltpu.async_copy(cur, out_hbm.at[pl.ds(off, SUPK)], sem_w)

    def body(s, carry):
        @pl.when(s % 2 == 0)
        def _():
            do_super(s, buf0, buf1, sem_g0, sem_g1)

        @pl.when(s % 2 == 1)
        def _():
            do_super(s, buf1, buf0, sem_g1, sem_g0)

        return carry

    lax.fori_loop(0, NSUP, body, 0, unroll=False)
    last = buf0 if (NSUP - 1) % 2 == 0 else buf1
    drain_wb(last)


def _sc_gather(g, src3, warm):
    mesh = plsc.VectorSubcoreMesh(core_axis_name="c", subcore_axis_name="s", num_cores=NC, num_subcores=NS)
    f = pl.kernel(
        _gather_body,
        out_type=jax.ShapeDtypeStruct((E, D), jnp.float32),
        mesh=mesh,
        scratch_types=[
            pltpu.VMEM((STEPS, K), jnp.int32),
            pltpu.VMEM((SUPK, D), jnp.float32),
            pltpu.VMEM((SUPK, D), jnp.float32),
            pltpu.SemaphoreType.DMA,
            pltpu.SemaphoreType.DMA,
            pltpu.SemaphoreType.DMA,
        ],
    )
    return f(g, src3, warm)


def _scatter_body(m_hbm, dst_hbm, zeros_hbm, out_hbm, idx0, idx1, buf0,
                  buf1, acc_sh, sem_l0, sem_l1, sem_a0, sem_a1):
    cid = lax.axis_index("c")
    sid = lax.axis_index("s")
    wid = sid * NC + cid
    zoff = pl.multiple_of(sid * ZR, 8)

    @pl.when(sid < NS - 1)
    def _zero_main():
        pltpu.sync_copy(zeros_hbm.at[pl.ds(zoff, ZR)],
                        acc_sh.at[pl.ds(zoff, ZR)])

    @pl.when(sid == NS - 1)
    def _zero_tail():
        pltpu.sync_copy(zeros_hbm.at[pl.ds(zoff, ZR_LAST)],
                        acc_sh.at[pl.ds(zoff, ZR_LAST)])

    base = wid * EPW
    plsc.subcore_barrier()

    def issue_load(i, buf, idx, sem):
        off = pl.multiple_of(base + i * SROWS, 8)
        pltpu.async_copy(m_hbm.at[pl.ds(off, SROWS)], buf, sem)
        pltpu.async_copy(dst_hbm.at[wid, i], idx, sem)

    def wait_load(buf, idx, sem):
        pltpu.make_async_copy(m_hbm.at[pl.ds(0, SROWS)], buf, sem).wait()
        pltpu.make_async_copy(dst_hbm.at[wid, 0], idx, sem).wait()

    def issue_adds(buf, idx, sem):
        for b in range(SG):
            pltpu.async_copy(buf.at[pl.ds(b * SK, SK)],
                             acc_sh.at[idx.at[b]], sem, add=True)

    def drain_adds(buf, idx, sem):
        for b in range(SG):
            pltpu.make_async_copy(buf.at[pl.ds(b * SK, SK)],
                                  acc_sh.at[idx.at[b]], sem).wait()

    issue_load(0, buf0, idx0, sem_l0)

    def do_group(i, cur, nxt, icur, inxt, sem_l_cur, sem_l_nxt, sem_a_cur,
                 sem_a_nxt):
        @pl.when(i >= 1)
        def _():
            drain_adds(nxt, inxt, sem_a_nxt)

        @pl.when(i + 1 < SGROUPS)
        def _():
            issue_load(i + 1, nxt, inxt, sem_l_nxt)

        wait_load(cur, icur, sem_l_cur)
        issue_adds(cur, icur, sem_a_cur)

    def body(i, carry):
        @pl.when(i % 2 == 0)
        def _():
            do_group(i, buf0, buf1, idx0, idx1, sem_l0, sem_l1, sem_a0,
                     sem_a1)

        @pl.when(i % 2 == 1)
        def _():
            do_group(i, buf1, buf0, idx1, idx0, sem_l1, sem_l0, sem_a1,
                     sem_a0)

        return carry

    lax.fori_loop(0, SGROUPS, body, 0, unroll=False)
    if (SGROUPS - 1) % 2 == 0:
        drain_adds(buf0, idx0, sem_a0)
    else:
        drain_adds(buf1, idx1, sem_a1)
    plsc.subcore_barrier()

    @pl.when(sid < NS - 1)
    def _out_main():
        pltpu.sync_copy(acc_sh.at[pl.ds(zoff, ZR)],
                        out_hbm.at[cid, pl.ds(zoff, ZR)])

    @pl.when(sid == NS - 1)
    def _out_tail():
        pltpu.sync_copy(acc_sh.at[pl.ds(zoff, ZR_LAST)],
                        out_hbm.at[cid, pl.ds(zoff, ZR_LAST)])


def _sc_scatter(m, dst3, zeros_nd):
    mesh = plsc.VectorSubcoreMesh(core_axis_name="c", subcore_axis_name="s", num_cores=NC, num_subcores=NS)
    f = pl.kernel(
        _scatter_body,
        out_type=jax.ShapeDtypeStruct((NC, N, D), jnp.float32),
        mesh=mesh,
        scratch_types=[
            pltpu.VMEM((SG, SK), jnp.int32),
            pltpu.VMEM((SG, SK), jnp.int32),
            pltpu.VMEM((SROWS, D), jnp.float32),
            pltpu.VMEM((SROWS, D), jnp.float32),
            pltpu.VMEM_SHARED((N, D), jnp.float32),
            pltpu.SemaphoreType.DMA,
            pltpu.SemaphoreType.DMA,
            pltpu.SemaphoreType.DMA,
            pltpu.SemaphoreType.DMA,
        ],
    )
    return f(m, dst3, zeros_nd)


def _embed_body(xs, xe, ws, we, b, w1a, b1, oh, og):
    h = jnp.maximum(xs[...] @ ws[...] + xe[...] @ we[...] + b[...], 0.0)
    oh[...] = h
    og[...] = h @ w1a[...] + b1[...]


def _tc_embed(x_struct, x_esm, ws, we, b, w1a, b1):
    return pl.pallas_call(
        _embed_body,
        grid=(N // BN,),
        in_specs=[
            pl.BlockSpec((BN, D), lambda i: (i, 0)),
            pl.BlockSpec((BN, D), lambda i: (i, 0)),
            pl.BlockSpec((D, D), lambda i: (0, 0)),
            pl.BlockSpec((D, D), lambda i: (0, 0)),
            pl.BlockSpec((1, D), lambda i: (0, 0)),
            pl.BlockSpec((D, D), lambda i: (0, 0)),
            pl.BlockSpec((1, D), lambda i: (0, 0)),
        ],
        out_specs=[pl.BlockSpec((BN, D), lambda i: (i, 0)),
                   pl.BlockSpec((BN, D), lambda i: (i, 0))],
        out_shape=[jax.ShapeDtypeStruct((N, D), jnp.float32),
                   jax.ShapeDtypeStruct((N, D), jnp.float32)],
    )(x_struct, x_esm, ws, we, b, w1a, b1)


def _edge_mlp_body(hg, ea, w1b, w2, b2, o):
    z = jnp.maximum(hg[...] + ea[...] @ w1b[...], 0.0)
    zb = z.astype(jnp.bfloat16)
    w2b = w2[...].astype(jnp.bfloat16)
    acc = jax.lax.dot_general(zb, w2b, (((1,), (0,)), ((), ())),
                              preferred_element_type=jnp.float32)
    o[...] = jnp.maximum(acc + b2[...], 0.0)


def _tc_edge_mlp(hg, ea, w1b, w2, b2):
    return pl.pallas_call(
        _edge_mlp_body,
        grid=(E // BE,),
        in_specs=[
            pl.BlockSpec((BE, D), lambda i: (i, 0)),
            pl.BlockSpec((BE, ED), lambda i: (i, 0)),
            pl.BlockSpec((ED, D), lambda i: (0, 0)),
            pl.BlockSpec((D, D), lambda i: (0, 0)),
            pl.BlockSpec((1, D), lambda i: (0, 0)),
        ],
        out_specs=pl.BlockSpec((BE, D), lambda i: (i, 0)),
        out_shape=jax.ShapeDtypeStruct((E, D), jnp.float32),
    )(hg, ea, w1b, w2, b2)


def _update_g_body(h, p, wh, wa, b, lng, lnb, w1a, b1, oh, og):
    agg = p[0] + p[1]
    hn = jnp.maximum(h[...] @ wh[...] + agg @ wa[...] + b[...], 0.0)
    r = hn + h[...]
    mu = jnp.mean(r, axis=-1, keepdims=True)
    var = jnp.mean((r - mu) ** 2, axis=-1, keepdims=True)
    hout = (r - mu) * lax.rsqrt(var + 1e-5) * lng[...] + lnb[...]
    oh[...] = hout
    og[...] = hout @ w1a[...] + b1[...]


def _tc_update_g(h, parts, wh, wa, b, lng, lnb, w1a, b1):
    return pl.pallas_call(
        _update_g_body,
        grid=(N // BN,),
        in_specs=[
            pl.BlockSpec((BN, D), lambda i: (i, 0)),
            pl.BlockSpec((NC, BN, D), lambda i: (0, i, 0)),
            pl.BlockSpec((D, D), lambda i: (0, 0)),
            pl.BlockSpec((D, D), lambda i: (0, 0)),
            pl.BlockSpec((1, D), lambda i: (0, 0)),
            pl.BlockSpec((1, D), lambda i: (0, 0)),
            pl.BlockSpec((1, D), lambda i: (0, 0)),
            pl.BlockSpec((D, D), lambda i: (0, 0)),
            pl.BlockSpec((1, D), lambda i: (0, 0)),
        ],
        out_specs=[pl.BlockSpec((BN, D), lambda i: (i, 0)),
                   pl.BlockSpec((BN, D), lambda i: (i, 0))],
        out_shape=[jax.ShapeDtypeStruct((N, D), jnp.float32),
                   jax.ShapeDtypeStruct((N, D), jnp.float32)],
    )(h, parts, wh, wa, b, lng, lnb, w1a, b1)


def _update_final_body(h, p, wh, wa, b, lng, lnb, fng, fnb, hw, hb, o):
    agg = p[0] + p[1]
    hn = jnp.maximum(h[...] @ wh[...] + agg @ wa[...] + b[...], 0.0)
    r = hn + h[...]
    mu = jnp.mean(r, axis=-1, keepdims=True)
    var = jnp.mean((r - mu) ** 2, axis=-1, keepdims=True)
    r = (r - mu) * lax.rsqrt(var + 1e-5) * lng[...] + lnb[...]
    mu = jnp.mean(r, axis=-1, keepdims=True)
    var = jnp.mean((r - mu) ** 2, axis=-1, keepdims=True)
    hn = (r - mu) * lax.rsqrt(var + 1e-5) * fng[...] + fnb[...]
    o[...] = hn @ hw[...] + hb[...]


def _tc_update_final(h, parts, wh, wa, b, lng, lnb, fng, fnb, hw, hb):
    return pl.pallas_call(
        _update_final_body,
        grid=(N // BN,),
        in_specs=[
            pl.BlockSpec((BN, D), lambda i: (i, 0)),
            pl.BlockSpec((NC, BN, D), lambda i: (0, i, 0)),
            pl.BlockSpec((D, D), lambda i: (0, 0)),
            pl.BlockSpec((D, D), lambda i: (0, 0)),
            pl.BlockSpec((1, D), lambda i: (0, 0)),
            pl.BlockSpec((1, D), lambda i: (0, 0)),
            pl.BlockSpec((1, D), lambda i: (0, 0)),
            pl.BlockSpec((1, D), lambda i: (0, 0)),
            pl.BlockSpec((1, D), lambda i: (0, 0)),
            pl.BlockSpec((D, C), lambda i: (0, 0)),
            pl.BlockSpec((1, C), lambda i: (0, 0)),
        ],
        out_specs=pl.BlockSpec((BN, C), lambda i: (i, 0)),
        out_shape=jax.ShapeDtypeStruct((N, C), jnp.float32),
    )(h, parts, wh, wa, b, lng, lnb, fng, fnb, hw, hb)


def _final_body(h, fng, fnb, hw, hb, o):
    r = h[...]
    mu = jnp.mean(r, axis=-1, keepdims=True)
    var = jnp.mean((r - mu) ** 2, axis=-1, keepdims=True)
    hn = (r - mu) * lax.rsqrt(var + 1e-5) * fng[...] + fnb[...]
    o[...] = hn @ hw[...] + hb[...]


def _tc_final(h, fng, fnb, hw, hb):
    return pl.pallas_call(
        _final_body,
        grid=(N // BN,),
        in_specs=[
            pl.BlockSpec((BN, D), lambda i: (i, 0)),
            pl.BlockSpec((1, D), lambda i: (0, 0)),
            pl.BlockSpec((1, D), lambda i: (0, 0)),
            pl.BlockSpec((D, C), lambda i: (0, 0)),
            pl.BlockSpec((1, C), lambda i: (0, 0)),
        ],
        out_specs=pl.BlockSpec((BN, C), lambda i: (i, 0)),
        out_shape=jax.ShapeDtypeStruct((N, C), jnp.float32),
    )(h, fng, fnb, hw, hb)


def kernel(x_struct, x_esm, edge_index, edge_attr,
           lin_struct_W, lin_struct_b, lin_esm_W, lin_esm_b,
           msg_W1, msg_b1, msg_W2, msg_b2,
           upd_W, upd_b, ln_g, ln_b, fn_g, fn_b,
           head_W, head_b):
    src3 = edge_index[0].reshape(NW, STEPS, K)
    dst3 = edge_index[1].reshape(NW, SGROUPS, SG, SK)
    zeros_nd = jnp.zeros((N, D), jnp.float32)
    b0 = (lin_struct_b + lin_esm_b).reshape(1, D)

    warm = _sc_warm(edge_index[0, :8])
    h, g = _tc_embed(x_struct, x_esm, lin_struct_W, lin_esm_W, b0,
                     msg_W1[0, :D, :], msg_b1[0].reshape(1, D))
    for l in range(L):
        w1b = msg_W1[l, D:, :]
        hg = _sc_gather(g, src3, warm)
        m = _tc_edge_mlp(hg, edge_attr, w1b, msg_W2[l],
                         msg_b2[l].reshape(1, D))
        parts = _sc_scatter(m, dst3, zeros_nd)
        if l < L - 1:
            h, g = _tc_update_g(h, parts, upd_W[l, :D, :], upd_W[l, D:, :],
                                upd_b[l].reshape(1, D), ln_g[l].reshape(1, D),
                                ln_b[l].reshape(1, D), msg_W1[l + 1, :D, :],
                                msg_b1[l + 1].reshape(1, D))
        else:
            return _tc_update_final(h, parts, upd_W[l, :D, :],
                                    upd_W[l, D:, :], upd_b[l].reshape(1, D),
                                    ln_g[l].reshape(1, D),
                                    ln_b[l].reshape(1, D),
                                    fn_g.reshape(1, D), fn_b.reshape(1, D),
                                    head_W, head_b.reshape(1, C))


# speedup vs baseline: 1.0023x; 1.0023x over previous
import functools

import jax
import jax.numpy as jnp
from jax import lax
from jax.experimental import pallas as pl
from jax.experimental.pallas import tpu as pltpu
from jax.experimental.pallas import tpu_sc as plsc

N = 10000
E = 320000
D = 128
ED = 16
L = 4
C = 20

NC = 2
NS = 16
NW = NC * NS
EPW = E // NW
K = 80
STEPS = EPW // K
NB = 5
SUPK = NB * K
NSUP = STEPS // NB
SK = 80
SSTEPS = EPW // SK
SG = 1
SROWS = SG * SK
SGROUPS = SSTEPS // SG
ZR = 624
ZR_LAST = N - (NS - 1) * ZR

BN = 2000
BE = 16000


def _gather_body(g_hbm, src_hbm, out_hbm, idx_v, buf0, buf1,
                 sem_g0, sem_g1, sem_w):
    wid = lax.axis_index("s") * NC + lax.axis_index("c")
    base = wid * EPW
    pltpu.sync_copy(src_hbm.at[wid], idx_v)

    def issue_gathers(s, buf, sem):
        for b in range(NB):
            j = s * NB + b
            pltpu.async_copy(g_hbm.at[idx_v.at[j]],
                             buf.at[pl.ds(b * K, K)], sem)

    def drain_gathers(buf, sem):
        for b in range(NB):
            pltpu.make_async_copy(g_hbm.at[idx_v.at[0]],
                                  buf.at[pl.ds(b * K, K)], sem).wait()

    def drain_wb(buf):
        pltpu.make_async_copy(buf, out_hbm.at[pl.ds(0, SUPK)], sem_w).wait()

    issue_gathers(0, buf0, sem_g0)

    def do_super(s, cur, nxt, sem_cur, sem_nxt):
        @pl.when(s >= 1)
        def _():
            drain_wb(nxt)

        @pl.when(s + 1 < NSUP)
        def _():
            issue_gathers(s + 1, nxt, sem_nxt)

        drain_gathers(cur, sem_cur)
        off = pl.multiple_of(base + s * SUPK, 8)
        pltpu.async_copy(cur, out_hbm.at[pl.ds(off, SUPK)], sem_w)

    def body(s, carry):
        @pl.when(s % 2 == 0)
        def _():
            do_super(s, buf0, buf1, sem_g0, sem_g1)

        @pl.when(s % 2 == 1)
        def _():
            do_super(s, buf1, buf0, sem_g1, sem_g0)

        return carry

    lax.fori_loop(0, NSUP, body, 0, unroll=False)
    last = buf0 if (NSUP - 1) % 2 == 0 else buf1
    drain_wb(last)


def _sc_gather(g, src3):
    mesh = plsc.VectorSubcoreMesh(core_axis_name="c", subcore_axis_name="s", num_cores=NC, num_subcores=NS)
    f = pl.kernel(
        _gather_body,
        out_type=jax.ShapeDtypeStruct((E, D), jnp.float32),
        mesh=mesh,
        scratch_types=[
            pltpu.VMEM((STEPS, K), jnp.int32),
            pltpu.VMEM((SUPK, D), jnp.float32),
            pltpu.VMEM((SUPK, D), jnp.float32),
            pltpu.SemaphoreType.DMA,
            pltpu.SemaphoreType.DMA,
            pltpu.SemaphoreType.DMA,
        ],
    )
    return f(g, src3)


def _scatter_body(m_hbm, dst_hbm, zeros_hbm, out_hbm, idx0, idx1, buf0,
                  buf1, acc_sh, sem_l0, sem_l1, sem_a0, sem_a1):
    cid = lax.axis_index("c")
    sid = lax.axis_index("s")
    wid = sid * NC + cid
    zoff = pl.multiple_of(sid * ZR, 8)

    @pl.when(sid < NS - 1)
    def _zero_main():
        pltpu.sync_copy(zeros_hbm.at[pl.ds(zoff, ZR)],
                        acc_sh.at[pl.ds(zoff, ZR)])

    @pl.when(sid == NS - 1)
    def _zero_tail():
        pltpu.sync_copy(zeros_hbm.at[pl.ds(zoff, ZR_LAST)],
                        acc_sh.at[pl.ds(zoff, ZR_LAST)])

    base = wid * EPW
    plsc.subcore_barrier()

    def issue_load(i, buf, idx, sem):
        off = pl.multiple_of(base + i * SROWS, 8)
        pltpu.async_copy(m_hbm.at[pl.ds(off, SROWS)], buf, sem)
        pltpu.async_copy(dst_hbm.at[wid, i], idx, sem)

    def wait_load(buf, idx, sem):
        pltpu.make_async_copy(m_hbm.at[pl.ds(0, SROWS)], buf, sem).wait()
        pltpu.make_async_copy(dst_hbm.at[wid, 0], idx, sem).wait()

    def issue_adds(buf, idx, sem):
        for b in range(SG):
            pltpu.async_copy(buf.at[pl.ds(b * SK, SK)],
                             acc_sh.at[idx.at[b]], sem, add=True)

    def drain_adds(buf, idx, sem):
        for b in range(SG):
            pltpu.make_async_copy(buf.at[pl.ds(b * SK, SK)],
                                  acc_sh.at[idx.at[b]], sem).wait()

    issue_load(0, buf0, idx0, sem_l0)

    def do_group(i, cur, nxt, icur, inxt, sem_l_cur, sem_l_nxt, sem_a_cur,
                 sem_a_nxt):
        @pl.when(i >= 1)
        def _():
            drain_adds(nxt, inxt, sem_a_nxt)

        @pl.when(i + 1 < SGROUPS)
        def _():
            issue_load(i + 1, nxt, inxt, sem_l_nxt)

        wait_load(cur, icur, sem_l_cur)
        issue_adds(cur, icur, sem_a_cur)

    def body(i, carry):
        @pl.when(i % 2 == 0)
        def _():
            do_group(i, buf0, buf1, idx0, idx1, sem_l0, sem_l1, sem_a0,
                     sem_a1)

        @pl.when(i % 2 == 1)
        def _():
            do_group(i, buf1, buf0, idx1, idx0, sem_l1, sem_l0, sem_a1,
                     sem_a0)

        return carry

    lax.fori_loop(0, SGROUPS, body, 0, unroll=False)
    if (SGROUPS - 1) % 2 == 0:
        drain_adds(buf0, idx0, sem_a0)
    else:
        drain_adds(buf1, idx1, sem_a1)
    plsc.subcore_barrier()

    @pl.when(sid < NS - 1)
    def _out_main():
        pltpu.sync_copy(acc_sh.at[pl.ds(zoff, ZR)],
                        out_hbm.at[cid, pl.ds(zoff, ZR)])

    @pl.when(sid == NS - 1)
    def _out_tail():
        pltpu.sync_copy(acc_sh.at[pl.ds(zoff, ZR_LAST)],
                        out_hbm.at[cid, pl.ds(zoff, ZR_LAST)])


def _sc_scatter(m, dst3, zeros_nd):
    mesh = plsc.VectorSubcoreMesh(core_axis_name="c", subcore_axis_name="s", num_cores=NC, num_subcores=NS)
    f = pl.kernel(
        _scatter_body,
        out_type=jax.ShapeDtypeStruct((NC, N, D), jnp.float32),
        mesh=mesh,
        scratch_types=[
            pltpu.VMEM((SG, SK), jnp.int32),
            pltpu.VMEM((SG, SK), jnp.int32),
            pltpu.VMEM((SROWS, D), jnp.float32),
            pltpu.VMEM((SROWS, D), jnp.float32),
            pltpu.VMEM_SHARED((N, D), jnp.float32),
            pltpu.SemaphoreType.DMA,
            pltpu.SemaphoreType.DMA,
            pltpu.SemaphoreType.DMA,
            pltpu.SemaphoreType.DMA,
        ],
    )
    return f(m, dst3, zeros_nd)


def _embed_body(xs, xe, ws, we, b, w1a, b1, oh, og):
    h = jnp.maximum(xs[...] @ ws[...] + xe[...] @ we[...] + b[...], 0.0)
    oh[...] = h
    og[...] = h @ w1a[...] + b1[...]


def _tc_embed(x_struct, x_esm, ws, we, b, w1a, b1):
    return pl.pallas_call(
        _embed_body,
        grid=(N // BN,),
        in_specs=[
            pl.BlockSpec((BN, D), lambda i: (i, 0)),
            pl.BlockSpec((BN, D), lambda i: (i, 0)),
            pl.BlockSpec((D, D), lambda i: (0, 0)),
            pl.BlockSpec((D, D), lambda i: (0, 0)),
            pl.BlockSpec((1, D), lambda i: (0, 0)),
            pl.BlockSpec((D, D), lambda i: (0, 0)),
            pl.BlockSpec((1, D), lambda i: (0, 0)),
        ],
        out_specs=[pl.BlockSpec((BN, D), lambda i: (i, 0)),
                   pl.BlockSpec((BN, D), lambda i: (i, 0))],
        out_shape=[jax.ShapeDtypeStruct((N, D), jnp.float32),
                   jax.ShapeDtypeStruct((N, D), jnp.float32)],
    )(x_struct, x_esm, ws, we, b, w1a, b1)


def _edge_mlp_body(hg, ea, w1b, w2, b2, o):
    z = jnp.maximum(hg[...] + ea[...] @ w1b[...], 0.0)
    zb = z.astype(jnp.bfloat16)
    w2b = w2[...].astype(jnp.bfloat16)
    acc = jax.lax.dot_general(zb, w2b, (((1,), (0,)), ((), ())),
                              preferred_element_type=jnp.float32)
    o[...] = jnp.maximum(acc + b2[...], 0.0)


def _tc_edge_mlp(hg, ea, w1b, w2, b2):
    return pl.pallas_call(
        _edge_mlp_body,
        grid=(E // BE,),
        in_specs=[
            pl.BlockSpec((BE, D), lambda i: (i, 0)),
            pl.BlockSpec((BE, ED), lambda i: (i, 0)),
            pl.BlockSpec((ED, D), lambda i: (0, 0)),
            pl.BlockSpec((D, D), lambda i: (0, 0)),
            pl.BlockSpec((1, D), lambda i: (0, 0)),
        ],
        out_specs=pl.BlockSpec((BE, D), lambda i: (i, 0)),
        out_shape=jax.ShapeDtypeStruct((E, D), jnp.float32),
    )(hg, ea, w1b, w2, b2)


def _update_g_body(h, p, wh, wa, b, lng, lnb, w1a, b1, oh, og):
    agg = p[0] + p[1]
    hn = jnp.maximum(h[...] @ wh[...] + agg @ wa[...] + b[...], 0.0)
    r = hn + h[...]
    mu = jnp.mean(r, axis=-1, keepdims=True)
    var = jnp.mean((r - mu) ** 2, axis=-1, keepdims=True)
    hout = (r - mu) * lax.rsqrt(var + 1e-5) * lng[...] + lnb[...]
    oh[...] = hout
    og[...] = hout @ w1a[...] + b1[...]


def _tc_update_g(h, parts, wh, wa, b, lng, lnb, w1a, b1):
    return pl.pallas_call(
        _update_g_body,
        grid=(N // BN,),
        in_specs=[
            pl.BlockSpec((BN, D), lambda i: (i, 0)),
            pl.BlockSpec((NC, BN, D), lambda i: (0, i, 0)),
            pl.BlockSpec((D, D), lambda i: (0, 0)),
            pl.BlockSpec((D, D), lambda i: (0, 0)),
            pl.BlockSpec((1, D), lambda i: (0, 0)),
            pl.BlockSpec((1, D), lambda i: (0, 0)),
            pl.BlockSpec((1, D), lambda i: (0, 0)),
            pl.BlockSpec((D, D), lambda i: (0, 0)),
            pl.BlockSpec((1, D), lambda i: (0, 0)),
        ],
        out_specs=[pl.BlockSpec((BN, D), lambda i: (i, 0)),
                   pl.BlockSpec((BN, D), lambda i: (i, 0))],
        out_shape=[jax.ShapeDtypeStruct((N, D), jnp.float32),
                   jax.ShapeDtypeStruct((N, D), jnp.float32)],
    )(h, parts, wh, wa, b, lng, lnb, w1a, b1)


def _update_final_body(h, p, wh, wa, b, lng, lnb, fng, fnb, hw, hb, o):
    agg = p[0] + p[1]
    hn = jnp.maximum(h[...] @ wh[...] + agg @ wa[...] + b[...], 0.0)
    r = hn + h[...]
    mu = jnp.mean(r, axis=-1, keepdims=True)
    var = jnp.mean((r - mu) ** 2, axis=-1, keepdims=True)
    r = (r - mu) * lax.rsqrt(var + 1e-5) * lng[...] + lnb[...]
    mu = jnp.mean(r, axis=-1, keepdims=True)
    var = jnp.mean((r - mu) ** 2, axis=-1, keepdims=True)
    hn = (r - mu) * lax.rsqrt(var + 1e-5) * fng[...] + fnb[...]
    o[...] = hn @ hw[...] + hb[...]


def _tc_update_final(h, parts, wh, wa, b, lng, lnb, fng, fnb, hw, hb):
    return pl.pallas_call(
        _update_final_body,
        grid=(N // BN,),
        in_specs=[
            pl.BlockSpec((BN, D), lambda i: (i, 0)),
            pl.BlockSpec((NC, BN, D), lambda i: (0, i, 0)),
            pl.BlockSpec((D, D), lambda i: (0, 0)),
            pl.BlockSpec((D, D), lambda i: (0, 0)),
            pl.BlockSpec((1, D), lambda i: (0, 0)),
            pl.BlockSpec((1, D), lambda i: (0, 0)),
            pl.BlockSpec((1, D), lambda i: (0, 0)),
            pl.BlockSpec((1, D), lambda i: (0, 0)),
            pl.BlockSpec((1, D), lambda i: (0, 0)),
            pl.BlockSpec((D, C), lambda i: (0, 0)),
            pl.BlockSpec((1, C), lambda i: (0, 0)),
        ],
        out_specs=pl.BlockSpec((BN, C), lambda i: (i, 0)),
        out_shape=jax.ShapeDtypeStruct((N, C), jnp.float32),
    )(h, parts, wh, wa, b, lng, lnb, fng, fnb, hw, hb)


def _final_body(h, fng, fnb, hw, hb, o):
    r = h[...]
    mu = jnp.mean(r, axis=-1, keepdims=True)
    var = jnp.mean((r - mu) ** 2, axis=-1, keepdims=True)
    hn = (r - mu) * lax.rsqrt(var + 1e-5) * fng[...] + fnb[...]
    o[...] = hn @ hw[...] + hb[...]


def _tc_final(h, fng, fnb, hw, hb):
    return pl.pallas_call(
        _final_body,
        grid=(N // BN,),
        in_specs=[
            pl.BlockSpec((BN, D), lambda i: (i, 0)),
            pl.BlockSpec((1, D), lambda i: (0, 0)),
            pl.BlockSpec((1, D), lambda i: (0, 0)),
            pl.BlockSpec((D, C), lambda i: (0, 0)),
            pl.BlockSpec((1, C), lambda i: (0, 0)),
        ],
        out_specs=pl.BlockSpec((BN, C), lambda i: (i, 0)),
        out_shape=jax.ShapeDtypeStruct((N, C), jnp.float32),
    )(h, fng, fnb, hw, hb)


def kernel(x_struct, x_esm, edge_index, edge_attr,
           lin_struct_W, lin_struct_b, lin_esm_W, lin_esm_b,
           msg_W1, msg_b1, msg_W2, msg_b2,
           upd_W, upd_b, ln_g, ln_b, fn_g, fn_b,
           head_W, head_b):
    src3 = edge_index[0].reshape(NW, STEPS, K)
    dst3 = edge_index[1].reshape(NW, SGROUPS, SG, SK)
    zeros_nd = jnp.zeros((N, D), jnp.float32)
    b0 = (lin_struct_b + lin_esm_b).reshape(1, D)

    h, g = _tc_embed(x_struct, x_esm, lin_struct_W, lin_esm_W, b0,
                     msg_W1[0, :D, :], msg_b1[0].reshape(1, D))
    for l in range(L):
        w1b = msg_W1[l, D:, :]
        hg = _sc_gather(g, src3)
        m = _tc_edge_mlp(hg, edge_attr, w1b, msg_W2[l],
                         msg_b2[l].reshape(1, D))
        parts = _sc_scatter(m, dst3, zeros_nd)
        if l < L - 1:
            h, g = _tc_update_g(h, parts, upd_W[l, :D, :], upd_W[l, D:, :],
                                upd_b[l].reshape(1, D), ln_g[l].reshape(1, D),
                                ln_b[l].reshape(1, D), msg_W1[l + 1, :D, :],
                                msg_b1[l + 1].reshape(1, D))
        else:
            return _tc_update_final(h, parts, upd_W[l, :D, :],
                                    upd_W[l, D:, :], upd_b[l].reshape(1, D),
                                    ln_g[l].reshape(1, D),
                                    ln_b[l].reshape(1, D),
                                    fn_g.reshape(1, D), fn_b.reshape(1, D),
                                    head_W, head_b.reshape(1, C))
